# baseline (speedup 1.0000x reference)
import functools

import numpy as np
import jax
import jax.numpy as jnp
from jax import lax
from jax.experimental import pallas as pl
from jax.experimental.pallas import tpu as pltpu
from jax.experimental.pallas import tpu_sc as plsc

_L = 16
_F = 2
_T = 16384
_BASE_RES = 16
_SCALE = 1.4472692012786865
_RES = [int(np.floor(_BASE_RES * (_SCALE ** l))) for l in range(_L)]
_P2 = -1640531535
_P3 = 805459861
_CORNERS = ((0, 0, 0), (0, 0, 1), (0, 1, 0), (0, 1, 1),
            (1, 0, 0), (1, 0, 1), (1, 1, 0), (1, 1, 1))
_MIN_ROUGH, _MAX_ROUGH = 0.08, 1.0

_LANES = 16
_UNROLL = 2
_DROWS = 2 * _L + 4


def _sc_encode(x_t, tblp, *, interpret=False):
    n = x_t.shape[1]
    nc, ns = 2, 16
    nw = nc * ns
    pt = n // nw
    assert pt * nw == n and pt % _LANES == 0
    c = min(pt, 8192)
    nchunk = pt // c
    assert nchunk * c == pt

    mesh = plsc.VectorSubcoreMesh(core_axis_name="c", subcore_axis_name="s",
                                  num_cores=nc, num_subcores=ns)

    @functools.partial(
        pl.kernel,
        out_type=jax.ShapeDtypeStruct((_DROWS, n), jnp.float32),
        mesh=mesh,
        scratch_types=[
            pltpu.VMEM((3, c), jnp.float32),
            pltpu.VMEM((_T,), jnp.int32),
            pltpu.VMEM((_T,), jnp.int32),
            pltpu.VMEM((2, c), jnp.float32),
            pltpu.VMEM((2, c), jnp.float32),
            pltpu.VMEM((4, c), jnp.float32),
            pltpu.SemaphoreType.DMA,
            pltpu.SemaphoreType.DMA,
            pltpu.SemaphoreType.DMA,
            pltpu.SemaphoreType.DMA,
            pltpu.SemaphoreType.DMA,
        ],
        compiler_params=pltpu.CompilerParams(needs_layout_passes=False),
        interpret=interpret,
    )
    def enc_kernel(x_hbm, tp_hbm, out_hbm, xbuf, tb0, tb1, st0, st1, xs,
                   tsem0, tsem1, ssem0, ssem1, xsem):
        tsem = (tsem0, tsem1)
        ssem = (ssem0, ssem1)
        stg = (st0, st1)
        tbb = (tb0, tb1)
        wid = lax.axis_index("s") * nc + lax.axis_index("c")
        base = wid * pt

        def chunk_body(k, carry):
            cb = base + k * c
            pltpu.async_copy(tp_hbm.at[pl.ds(0, _T)], tbb[0], tsem[0])
            pltpu.sync_copy(x_hbm.at[:, pl.ds(cb, c)], xbuf)
            for l in range(_L):
                res = _RES[l]
                stride = res + 2
                dense = stride ** 3 <= _T
                b = l & 1
                tb = tbb[b]
                stag = stg[b]
                pltpu.make_async_copy(
                    tp_hbm.at[pl.ds(l * _T, _T)], tb, tsem[b]).wait()
                if l + 1 < _L:
                    pltpu.async_copy(tp_hbm.at[pl.ds((l + 1) * _T, _T)],
                                     tbb[1 - b], tsem[1 - b])
                if l >= 2:
                    pltpu.make_async_copy(
                        stag,
                        out_hbm.at[pl.ds(2 * (l - 2), 2), pl.ds(cb, c)],
                        ssem[b]).wait()

                def vsub(s, res, stride, dense, emit_x,
                         tb=tb, stag=stag):
                    xv = xbuf[0, s]
                    yv = xbuf[1, s]
                    zv = xbuf[2, s]
                    if emit_x:
                        xs[0, s] = xv + xv - 1.0
                        xs[1, s] = yv + yv - 1.0
                        xs[2, s] = zv + zv - 1.0
                        xs[3, s] = jnp.zeros((_LANES,), jnp.float32)
                    rf = jnp.float32(res)
                    px = xv * rf + 0.5
                    py = yv * rf + 0.5
                    pz = zv * rf + 0.5
                    cx = px.astype(jnp.int32)
                    cy = py.astype(jnp.int32)
                    cz = pz.astype(jnp.int32)
                    wx = px - cx.astype(jnp.float32)
                    wy = py - cy.astype(jnp.float32)
                    wz = pz - cz.astype(jnp.float32)
                    if dense:
                        s2 = stride * stride
                        hy0 = cy * stride
                        hz0 = cz * s2
                        hx = (cx, cx + 1)
                        hy = (hy0, hy0 + stride)
                        hz = (hz0, hz0 + s2)

                        def cidx(a, b, cc):
                            return hx[a] + hy[b] + hz[cc]
                    else:
                        hy0 = cy * _P2
                        hz0 = cz * _P3
                        hx = (cx, cx + 1)
                        hy = (hy0, hy0 + _P2)
                        hz = (hz0, hz0 + _P3)

                        def cidx(a, b, cc):
                            return (hx[a] ^ hy[b] ^ hz[cc]) & (_T - 1)

                    ux = 1.0 - wx
                    uy = 1.0 - wy
                    uz = 1.0 - wz
                    wab = {(0, 0): ux * uy, (0, 1): ux * wy,
                           (1, 0): wx * uy, (1, 1): wx * wy}
                    f0 = jnp.zeros((_LANES,), jnp.float32)
                    f1 = jnp.zeros((_LANES,), jnp.float32)
                    for (a, b, cc) in _CORNERS:
                        idx = cidx(a, b, cc)
                        wt = wab[(a, b)] * (wz if cc else uz)
                        g = plsc.load_gather(tb, [idx])
                        g0 = plsc.bitcast(g & jnp.int32(-65536), jnp.float32)
                        g1 = plsc.bitcast(g << 16, jnp.float32)
                        f0 = f0 + wt * g0
                        f1 = f1 + wt * g1
                    stag[0, s] = f0
                    stag[1, s] = f1

                def vbody(i, cr, res=res, stride=stride, dense=dense,
                          emit_x=(l == 0)):
                    for u in range(_UNROLL):
                        off = i * (_LANES * _UNROLL) + u * _LANES
                        vsub(pl.ds(off, _LANES), res, stride, dense, emit_x)
                    return cr

                lax.fori_loop(0, c // (_LANES * _UNROLL), vbody, 0)
                pltpu.async_copy(
                    stag, out_hbm.at[pl.ds(2 * l, 2), pl.ds(cb, c)], ssem[b])
                if l == 0:
                    pltpu.async_copy(
                        xs, out_hbm.at[pl.ds(2 * _L, 4), pl.ds(cb, c)], xsem)
            for l in (_L - 2, _L - 1):
                pltpu.make_async_copy(
                    stg[l & 1],
                    out_hbm.at[pl.ds(2 * l, 2), pl.ds(cb, c)],
                    ssem[l & 1]).wait()
            pltpu.make_async_copy(
                xs, out_hbm.at[pl.ds(2 * _L, 4), pl.ds(cb, c)], xsem).wait()
            return carry

        lax.fori_loop(0, nchunk, chunk_body, 0)

    return enc_kernel(x_t, tblp)


def _mlp_body(in_ref, w1_ref, w2_ref, w3_ref, diff_ref, spec_ref, rough_ref):
    h = jnp.dot(w1_ref[...], in_ref[...], preferred_element_type=jnp.float32)
    h = jnp.maximum(h, 0.0)
    h = jnp.maximum(
        jnp.dot(w2_ref[...], h, preferred_element_type=jnp.float32), 0.0)
    o = jnp.dot(w3_ref[...], h, preferred_element_type=jnp.float32)
    diff_ref[...] = jax.nn.sigmoid(o[0:3])
    spec_ref[...] = 1.0 - jax.nn.sigmoid(o[3:4])
    r = jax.nn.sigmoid(o[4:5])
    rough_ref[...] = r * _MIN_ROUGH + (1.0 - r) * _MAX_ROUGH


def _mlp(inall, w1p, w2t, w3, *, interpret=False):
    n = inall.shape[1]
    b = min(n, 4096)
    grid = (n // b,)
    f32 = jnp.float32
    return pl.pallas_call(
        _mlp_body,
        grid=grid,
        in_specs=[
            pl.BlockSpec((_DROWS, b), lambda j: (0, j)),
            pl.BlockSpec(w1p.shape, lambda j: (0, 0)),
            pl.BlockSpec(w2t.shape, lambda j: (0, 0)),
            pl.BlockSpec(w3.shape, lambda j: (0, 0)),
        ],
        out_specs=[
            pl.BlockSpec((3, b), lambda j: (0, j)),
            pl.BlockSpec((1, b), lambda j: (0, j)),
            pl.BlockSpec((1, b), lambda j: (0, j)),
        ],
        out_shape=[
            jax.ShapeDtypeStruct((3, n), f32),
            jax.ShapeDtypeStruct((1, n), f32),
            jax.ShapeDtypeStruct((1, n), f32),
        ],
        interpret=interpret,
    )(inall, w1p, w2t, w3)


def kernel(x, table, W1, W2, W3):
    w0 = lax.bitcast_convert_type(
        table[:, :, 0].astype(jnp.bfloat16), jnp.uint16).astype(jnp.uint32)
    w1 = lax.bitcast_convert_type(
        table[:, :, 1].astype(jnp.bfloat16), jnp.uint16).astype(jnp.uint32)
    tblp = lax.bitcast_convert_type(
        (w0 << 16) | w1, jnp.int32).reshape(-1)
    w1p = jnp.concatenate(
        [W1[3:], W1[:3], jnp.zeros((1, W1.shape[1]), W1.dtype)], axis=0).T
    w2t, w3t = W2.T, W3.T
    n = x.shape[0]
    nsplit = 16
    half = n // nsplit
    outs = []
    for hh in range(nsplit):
        xh = lax.slice(x, (hh * half, 0), ((hh + 1) * half, 3)).T
        inall = _sc_encode(xh, tblp)
        outs.append(_mlp(inall, w1p, w2t, w3t))
    diff = jnp.concatenate([o[0].T for o in outs], axis=0)
    spec = jnp.concatenate([o[1].T for o in outs], axis=0)
    rough = jnp.concatenate([o[2].T for o in outs], axis=0)
    return diff, spec, rough


# speedup vs baseline: 1.0949x; 1.0949x over previous
import functools

import numpy as np
import jax
import jax.numpy as jnp
from jax import lax
from jax.experimental import pallas as pl
from jax.experimental.pallas import tpu as pltpu
from jax.experimental.pallas import tpu_sc as plsc

_L = 16
_F = 2
_T = 16384
_BASE_RES = 16
_SCALE = 1.4472692012786865
_RES = [int(np.floor(_BASE_RES * (_SCALE ** l))) for l in range(_L)]
_P2 = -1640531535
_P3 = 805459861
_CORNERS = ((0, 0, 0), (0, 0, 1), (0, 1, 0), (0, 1, 1),
            (1, 0, 0), (1, 0, 1), (1, 1, 0), (1, 1, 1))
_MIN_ROUGH, _MAX_ROUGH = 0.08, 1.0

_LANES = 16
_UNROLL = 2
_DROWS = 2 * _L + 4


def _sc_encode(x_t, tblp, *, interpret=False):
    n = x_t.shape[1]
    nc, ns = 2, 16
    nw = nc * ns
    pt = n // nw
    assert pt * nw == n and pt % _LANES == 0
    c = min(pt, 8192)
    nchunk = pt // c
    assert nchunk * c == pt

    mesh = plsc.VectorSubcoreMesh(core_axis_name="c", subcore_axis_name="s",
                                  num_cores=nc, num_subcores=ns)

    @functools.partial(
        pl.kernel,
        out_type=jax.ShapeDtypeStruct((_DROWS, n), jnp.float32),
        mesh=mesh,
        scratch_types=[
            pltpu.VMEM((3, c), jnp.float32),
            pltpu.VMEM((_T,), jnp.int32),
            pltpu.VMEM((_T,), jnp.int32),
            pltpu.VMEM((2, c), jnp.float32),
            pltpu.VMEM((2, c), jnp.float32),
            pltpu.VMEM((4, c), jnp.float32),
            pltpu.SemaphoreType.DMA,
            pltpu.SemaphoreType.DMA,
            pltpu.SemaphoreType.DMA,
            pltpu.SemaphoreType.DMA,
            pltpu.SemaphoreType.DMA,
        ],
        compiler_params=pltpu.CompilerParams(needs_layout_passes=False),
        interpret=interpret,
    )
    def enc_kernel(x_hbm, tp_hbm, out_hbm, xbuf, tb0, tb1, st0, st1, xs,
                   tsem0, tsem1, ssem0, ssem1, xsem):
        tsem = (tsem0, tsem1)
        ssem = (ssem0, ssem1)
        stg = (st0, st1)
        tbb = (tb0, tb1)
        wid = lax.axis_index("s") * nc + lax.axis_index("c")
        base = wid * pt

        def chunk_body(k, carry):
            cb = base + k * c
            pltpu.async_copy(tp_hbm.at[pl.ds(0, _T)], tbb[0], tsem[0])
            pltpu.sync_copy(x_hbm.at[:, pl.ds(cb, c)], xbuf)
            for l in range(_L):
                res = _RES[l]
                stride = res + 2
                dense = stride ** 3 <= _T
                b = l & 1
                tb = tbb[b]
                stag = stg[b]
                pltpu.make_async_copy(
                    tp_hbm.at[pl.ds(l * _T, _T)], tb, tsem[b]).wait()
                if l + 1 < _L:
                    pltpu.async_copy(tp_hbm.at[pl.ds((l + 1) * _T, _T)],
                                     tbb[1 - b], tsem[1 - b])
                if l >= 2:
                    pltpu.make_async_copy(
                        stag,
                        out_hbm.at[pl.ds(2 * (l - 2), 2), pl.ds(cb, c)],
                        ssem[b]).wait()

                def vsub(s, res, stride, dense, emit_x,
                         tb=tb, stag=stag):
                    xv = xbuf[0, s]
                    yv = xbuf[1, s]
                    zv = xbuf[2, s]
                    if emit_x:
                        xs[0, s] = xv + xv - 1.0
                        xs[1, s] = yv + yv - 1.0
                        xs[2, s] = zv + zv - 1.0
                        xs[3, s] = jnp.zeros((_LANES,), jnp.float32)
                    rf = jnp.float32(res)
                    px = xv * rf + 0.5
                    py = yv * rf + 0.5
                    pz = zv * rf + 0.5
                    cx = px.astype(jnp.int32)
                    cy = py.astype(jnp.int32)
                    cz = pz.astype(jnp.int32)
                    wx = px - cx.astype(jnp.float32)
                    wy = py - cy.astype(jnp.float32)
                    wz = pz - cz.astype(jnp.float32)
                    if dense:
                        s2 = stride * stride
                        hy0 = cy * stride
                        hz0 = cz * s2
                        hx = (cx, cx + 1)
                        hy = (hy0, hy0 + stride)
                        hz = (hz0, hz0 + s2)

                        def cidx(a, b, cc):
                            return hx[a] + hy[b] + hz[cc]
                    else:
                        hy0 = cy * _P2
                        hz0 = cz * _P3
                        hx = (cx, cx + 1)
                        hy = (hy0, hy0 + _P2)
                        hz = (hz0, hz0 + _P3)

                        def cidx(a, b, cc):
                            return (hx[a] ^ hy[b] ^ hz[cc]) & (_T - 1)

                    ux = 1.0 - wx
                    uy = 1.0 - wy
                    uz = 1.0 - wz
                    wab = {(0, 0): ux * uy, (0, 1): ux * wy,
                           (1, 0): wx * uy, (1, 1): wx * wy}
                    f0 = jnp.zeros((_LANES,), jnp.float32)
                    f1 = jnp.zeros((_LANES,), jnp.float32)
                    for (a, b, cc) in _CORNERS:
                        idx = cidx(a, b, cc)
                        wt = wab[(a, b)] * (wz if cc else uz)
                        g = plsc.load_gather(tb, [idx])
                        g0 = plsc.bitcast(g & jnp.int32(-65536), jnp.float32)
                        g1 = plsc.bitcast(g << 16, jnp.float32)
                        f0 = f0 + wt * g0
                        f1 = f1 + wt * g1
                    stag[0, s] = f0
                    stag[1, s] = f1

                def vbody(i, cr, res=res, stride=stride, dense=dense,
                          emit_x=(l == 0)):
                    for u in range(_UNROLL):
                        off = i * (_LANES * _UNROLL) + u * _LANES
                        vsub(pl.ds(off, _LANES), res, stride, dense, emit_x)
                    return cr

                lax.fori_loop(0, c // (_LANES * _UNROLL), vbody, 0)
                pltpu.async_copy(
                    stag, out_hbm.at[pl.ds(2 * l, 2), pl.ds(cb, c)], ssem[b])
                if l == 0:
                    pltpu.async_copy(
                        xs, out_hbm.at[pl.ds(2 * _L, 4), pl.ds(cb, c)], xsem)
            for l in (_L - 2, _L - 1):
                pltpu.make_async_copy(
                    stg[l & 1],
                    out_hbm.at[pl.ds(2 * l, 2), pl.ds(cb, c)],
                    ssem[l & 1]).wait()
            pltpu.make_async_copy(
                xs, out_hbm.at[pl.ds(2 * _L, 4), pl.ds(cb, c)], xsem).wait()
            return carry

        lax.fori_loop(0, nchunk, chunk_body, 0)

    return enc_kernel(x_t, tblp)


def _mlp_body(in_ref, w1_ref, w2_ref, w3_ref, diff_ref, spec_ref, rough_ref):
    h = jnp.dot(w1_ref[...], in_ref[...], preferred_element_type=jnp.float32)
    h = jnp.maximum(h, 0.0)
    h = jnp.maximum(
        jnp.dot(w2_ref[...], h, preferred_element_type=jnp.float32), 0.0)
    o = jnp.dot(w3_ref[...], h, preferred_element_type=jnp.float32)
    diff_ref[...] = jax.nn.sigmoid(o[0:3])
    spec_ref[...] = 1.0 - jax.nn.sigmoid(o[3:4])
    r = jax.nn.sigmoid(o[4:5])
    rough_ref[...] = r * _MIN_ROUGH + (1.0 - r) * _MAX_ROUGH


def _mlp(inall, w1p, w2t, w3, *, interpret=False):
    n = inall.shape[1]
    b = min(n, 4096)
    grid = (n // b,)
    f32 = jnp.float32
    return pl.pallas_call(
        _mlp_body,
        grid=grid,
        in_specs=[
            pl.BlockSpec((_DROWS, b), lambda j: (0, j)),
            pl.BlockSpec(w1p.shape, lambda j: (0, 0)),
            pl.BlockSpec(w2t.shape, lambda j: (0, 0)),
            pl.BlockSpec(w3.shape, lambda j: (0, 0)),
        ],
        out_specs=[
            pl.BlockSpec((3, b), lambda j: (0, j)),
            pl.BlockSpec((1, b), lambda j: (0, j)),
            pl.BlockSpec((1, b), lambda j: (0, j)),
        ],
        out_shape=[
            jax.ShapeDtypeStruct((3, n), f32),
            jax.ShapeDtypeStruct((1, n), f32),
            jax.ShapeDtypeStruct((1, n), f32),
        ],
        interpret=interpret,
    )(inall, w1p, w2t, w3)


def kernel(x, table, W1, W2, W3):
    w0 = lax.bitcast_convert_type(
        table[:, :, 0].astype(jnp.bfloat16), jnp.uint16).astype(jnp.uint32)
    w1 = lax.bitcast_convert_type(
        table[:, :, 1].astype(jnp.bfloat16), jnp.uint16).astype(jnp.uint32)
    tblp = lax.bitcast_convert_type(
        (w0 << 16) | w1, jnp.int32).reshape(-1)
    w1p = jnp.concatenate(
        [W1[3:], W1[:3], jnp.zeros((1, W1.shape[1]), W1.dtype)], axis=0).T
    w2t, w3t = W2.T, W3.T
    n = x.shape[0]
    nsplit = 8
    half = n // nsplit
    outs = []
    for hh in range(nsplit):
        xh = lax.slice(x, (hh * half, 0), ((hh + 1) * half, 3)).T
        inall = _sc_encode(xh, tblp)
        outs.append(_mlp(inall, w1p, w2t, w3t))
    diff = jnp.concatenate([o[0].T for o in outs], axis=0)
    spec = jnp.concatenate([o[1].T for o in outs], axis=0)
    rough = jnp.concatenate([o[2].T for o in outs], axis=0)
    return diff, spec, rough
